# baseline (speedup 1.0000x reference)
import functools

import jax
import jax.numpy as jnp
from jax import lax
from jax.experimental import pallas as pl
from jax.experimental.pallas import tpu as pltpu
from jax.experimental.pallas import tpu_sc as plsc

NC = 2
NS = 16
NW = NC * NS
CHUNK = 2048
DIM = 16
LANES = 16


def _pad_to(n, mult):
    return ((n + mult - 1) // mult) * mult


@functools.lru_cache(maxsize=None)
def _make_gather(n_atom_pad, n_edge_pad, atom_rows, bond_rows):
    a_per_w = n_atom_pad // NW
    e_per_w = n_edge_pad // NW
    a_chunks = a_per_w // CHUNK
    e_chunks = e_per_w // CHUNK

    mesh = plsc.VectorSubcoreMesh(core_axis_name="c", subcore_axis_name="s")

    @functools.partial(
        pl.kernel,
        out_type=(
            jax.ShapeDtypeStruct((n_atom_pad, DIM), jnp.float32),
            jax.ShapeDtypeStruct((n_edge_pad, DIM), jnp.float32),
        ),
        mesh=mesh,
        scratch_types=[
            pltpu.VMEM((atom_rows, DIM), jnp.float32),
            pltpu.VMEM((bond_rows, DIM), jnp.float32),
            pltpu.VMEM((CHUNK,), jnp.int32),
            pltpu.VMEM((CHUNK, DIM), jnp.float32),
        ],
        compiler_params=pltpu.CompilerParams(
            use_tc_tiling_on_sc=False, needs_layout_passes=False),
    )
    def gather_kernel(atab, xidx, btab, eidx, xout, eout,
                      atab_v, btab_v, idx_v, rows_v):
        wid = lax.axis_index("s") * NC + lax.axis_index("c")
        pltpu.sync_copy(atab, atab_v)
        pltpu.sync_copy(btab, btab_v)

        def run(tab_v, idxs, out, n_chunks, per_w):
            base0 = wid * per_w

            def chunk_body(k, carry):
                base = base0 + k * CHUNK
                pltpu.sync_copy(idxs.at[pl.ds(base, CHUNK)], idx_v)

                def row_body(j, carry2):
                    iv = idx_v[pl.ds(j * LANES, LANES)]
                    rows = j * LANES + lax.iota(jnp.int32, LANES)
                    for d in range(DIM):
                        cols = jnp.full((LANES,), d, jnp.int32)
                        vals = plsc.load_gather(tab_v, [iv, cols])
                        plsc.store_scatter(rows_v, [rows, cols], vals)
                    return carry2

                lax.fori_loop(0, CHUNK // LANES, row_body, 0)
                pltpu.sync_copy(rows_v, out.at[pl.ds(base, CHUNK)])
                return carry

            lax.fori_loop(0, n_chunks, chunk_body, 0)

        run(atab_v, xidx, xout, a_chunks, a_per_w)
        run(btab_v, eidx, eout, e_chunks, e_per_w)

    return gather_kernel


def kernel(x, edge_attr, atom_table, bond_table):
    n_atom = x.shape[0] * x.shape[1]
    n_edge = edge_attr.shape[0] * edge_attr.shape[1]
    n_atom_pad = _pad_to(n_atom, NW * CHUNK)
    n_edge_pad = _pad_to(n_edge, NW * CHUNK)

    xf = jnp.pad(x.reshape(-1).astype(jnp.int32), (0, n_atom_pad - n_atom))
    ef = jnp.pad(edge_attr.reshape(-1).astype(jnp.int32), (0, n_edge_pad - n_edge))

    gk = _make_gather(n_atom_pad, n_edge_pad,
                      atom_table.shape[0], bond_table.shape[0])
    xo, eo = gk(atom_table, xf, bond_table, ef)
    x_emb = xo[:n_atom].reshape(x.shape[0], x.shape[1], DIM)
    e_emb = eo[:n_edge].reshape(edge_attr.shape[0], edge_attr.shape[1], DIM)
    return (x_emb, e_emb)


# speedup vs baseline: 1.3930x; 1.3930x over previous
import functools

import jax
import jax.numpy as jnp
from jax import lax
from jax.experimental import pallas as pl
from jax.experimental.pallas import tpu as pltpu
from jax.experimental.pallas import tpu_sc as plsc

NC = 2
NS = 16
NW = NC * NS
CHUNK = 2048
DIM = 16
LANES = 16


@functools.lru_cache(maxsize=None)
def _make_gather(n_atom, n_edge, atom_rows, bond_rows):
    a_nch = -(-n_atom // CHUNK)
    e_nch = -(-n_edge // CHUNK)

    mesh = plsc.VectorSubcoreMesh(core_axis_name="c", subcore_axis_name="s")

    @functools.partial(
        pl.kernel,
        out_type=(
            jax.ShapeDtypeStruct((n_atom, DIM), jnp.float32),
            jax.ShapeDtypeStruct((n_edge, DIM), jnp.float32),
        ),
        mesh=mesh,
        scratch_types=[
            pltpu.VMEM((atom_rows, DIM), jnp.float32),
            pltpu.VMEM((bond_rows, DIM), jnp.float32),
            pltpu.VMEM((CHUNK,), jnp.int32),
            pltpu.VMEM((CHUNK, DIM), jnp.float32),
        ],
        compiler_params=pltpu.CompilerParams(
            use_tc_tiling_on_sc=False, needs_layout_passes=False),
    )
    def gather_kernel(atab, xidx, btab, eidx, xout, eout,
                      atab_v, btab_v, idx_v, rows_v):
        wid = lax.axis_index("s") * NC + lax.axis_index("c")
        pltpu.sync_copy(atab, atab_v)
        pltpu.sync_copy(btab, btab_v)

        def run(tab_v, idxs, out, nch, n):
            n_mine = (nch - wid + NW - 1) // NW

            def chunk_body(k, carry):
                c = wid + k * NW
                base = jnp.minimum(c * CHUNK, n - CHUNK)
                pltpu.sync_copy(idxs.at[pl.ds(base, CHUNK)], idx_v)

                def row_body(j, carry2):
                    iv = idx_v[pl.ds(j * LANES, LANES)]
                    rows = j * LANES + lax.iota(jnp.int32, LANES)
                    for d in range(DIM):
                        cols = jnp.full((LANES,), d, jnp.int32)
                        vals = plsc.load_gather(tab_v, [iv, cols])
                        plsc.store_scatter(rows_v, [rows, cols], vals)
                    return carry2

                lax.fori_loop(0, CHUNK // LANES, row_body, 0)
                pltpu.sync_copy(rows_v, out.at[pl.ds(base, CHUNK)])
                return carry

            lax.fori_loop(0, n_mine, chunk_body, 0)

        run(atab_v, xidx, xout, a_nch, n_atom)
        run(btab_v, eidx, eout, e_nch, n_edge)

    return gather_kernel


def kernel(x, edge_attr, atom_table, bond_table):
    n_atom = x.shape[0] * x.shape[1]
    n_edge = edge_attr.shape[0] * edge_attr.shape[1]

    xf = x.reshape(-1).astype(jnp.int32)
    ef = edge_attr.reshape(-1).astype(jnp.int32)

    gk = _make_gather(n_atom, n_edge, atom_table.shape[0], bond_table.shape[0])
    xo, eo = gk(atom_table, xf, bond_table, ef)
    x_emb = xo.reshape(x.shape[0], x.shape[1], DIM)
    e_emb = eo.reshape(edge_attr.shape[0], edge_attr.shape[1], DIM)
    return (x_emb, e_emb)


# speedup vs baseline: 8.5453x; 6.1346x over previous
import functools

import jax
import jax.numpy as jnp
from jax import lax
from jax.experimental import pallas as pl
from jax.experimental.pallas import tpu as pltpu
from jax.experimental.pallas import tpu_sc as plsc

NC = 2
NS = 16
NW = NC * NS
CHUNK = 2048
DIM = 16
LANES = 16
TILE_R = 8
TILE_C = 128


@functools.lru_cache(maxsize=None)
def _make_gather(n_atom_cols, n_edge_cols, n_feat_atom, n_feat_edge,
                 atom_rows, bond_rows):
    a_tc = n_atom_cols // TILE_C
    e_tc = n_edge_cols // TILE_C
    a_nch = -(-n_atom_cols // CHUNK)
    e_nch = -(-n_edge_cols // CHUNK)
    a_tasks = n_feat_atom * a_nch
    e_tasks = n_feat_edge * e_nch
    plane = CHUNK * TILE_R

    mesh = plsc.VectorSubcoreMesh(core_axis_name="c", subcore_axis_name="s")

    @functools.partial(
        pl.kernel,
        out_type=(
            jax.ShapeDtypeStruct((n_feat_atom * DIM * n_atom_cols,), jnp.float32),
            jax.ShapeDtypeStruct((n_feat_edge * DIM * n_edge_cols,), jnp.float32),
        ),
        mesh=mesh,
        scratch_types=[
            pltpu.VMEM((atom_rows, DIM), jnp.float32),
            pltpu.VMEM((bond_rows, DIM), jnp.float32),
            pltpu.VMEM((CHUNK,), jnp.int32),
            pltpu.VMEM((2, plane), jnp.float32),
        ],
        compiler_params=pltpu.CompilerParams(
            use_tc_tiling_on_sc=False, needs_layout_passes=False),
    )
    def gather_kernel(atab, xidx, btab, eidx, xout, eout,
                      atab_v, btab_v, idx_v, rows_v):
        wid = lax.axis_index("s") * NC + lax.axis_index("c")
        pltpu.sync_copy(atab, atab_v)
        pltpu.sync_copy(btab, btab_v)

        def run(tab_v, idxs, out, n_tasks, nch, ncols, ntc):
            n_mine = (n_tasks - wid + NW - 1) // NW

            def chunk_body(k, carry):
                task = wid + k * NW
                f = task // nch
                n0 = jnp.minimum((task % nch) * CHUNK, ncols - CHUNK)
                pltpu.sync_copy(idxs.at[pl.ds(f * ncols + n0, CHUNK)], idx_v)

                def row_body(j, carry2):
                    iv = idx_v[pl.ds(j * LANES, LANES)]
                    off = (j // 8) * (TILE_R * TILE_C) + (j % 8) * LANES
                    for d in range(DIM):
                        vals = plsc.load_gather(
                            tab_v, [iv, jnp.full((LANES,), d, jnp.int32)])
                        rows_v[d // TILE_R,
                               pl.ds(off + (d % TILE_R) * TILE_C, LANES)] = vals
                    return carry2

                lax.fori_loop(0, CHUNK // LANES, row_body, 0)
                for tr in range(2):
                    q0 = ((f * 2 + tr) * ntc + n0 // TILE_C) * (TILE_R * TILE_C)
                    pltpu.sync_copy(rows_v.at[tr], out.at[pl.ds(q0, plane)])
                return carry

            lax.fori_loop(0, n_mine, chunk_body, 0)

        run(atab_v, xidx, xout, a_tasks, a_nch, n_atom_cols, a_tc)
        run(btab_v, eidx, eout, e_tasks, e_nch, n_edge_cols, e_tc)

    return gather_kernel


def kernel(x, edge_attr, atom_table, bond_table):
    n_atom, f_atom = x.shape
    n_edge, f_edge = edge_attr.shape
    a_cols = -(-n_atom // TILE_C) * TILE_C
    e_cols = -(-n_edge // TILE_C) * TILE_C

    xt = jnp.pad(x.T.astype(jnp.int32), ((0, 0), (0, a_cols - n_atom)))
    et = jnp.pad(edge_attr.T.astype(jnp.int32), ((0, 0), (0, e_cols - n_edge)))

    gk = _make_gather(a_cols, e_cols, f_atom, f_edge,
                      atom_table.shape[0], bond_table.shape[0])
    xo, eo = gk(atom_table, xt.reshape(-1), bond_table, et.reshape(-1))

    x_emb = (xo.reshape(f_atom, 2, a_cols // TILE_C, TILE_R, TILE_C)
             .transpose(2, 4, 0, 1, 3)
             .reshape(a_cols, f_atom, DIM)[:n_atom])
    e_emb = (eo.reshape(f_edge, 2, e_cols // TILE_C, TILE_R, TILE_C)
             .transpose(2, 4, 0, 1, 3)
             .reshape(e_cols, f_edge, DIM)[:n_edge])
    return (x_emb, e_emb)


# speedup vs baseline: 20.4478x; 2.3929x over previous
import functools

import jax
import jax.numpy as jnp
from jax import lax
from jax.experimental import pallas as pl
from jax.experimental.pallas import tpu as pltpu
from jax.experimental.pallas import tpu_sc as plsc

NC = 2
NS = 16
NW = NC * NS
CHUNK = 2048
DIM = 16
LANES = 16
TILE_R = 8
TILE_C = 128


@functools.lru_cache(maxsize=None)
def _make_gather(n_atom_cols, n_edge_cols, n_feat_atom, n_feat_edge,
                 atom_rows, bond_rows):
    a_tc = n_atom_cols // TILE_C
    e_tc = n_edge_cols // TILE_C
    a_nch = -(-n_atom_cols // CHUNK)
    e_nch = -(-n_edge_cols // CHUNK)
    a_tasks = n_feat_atom * a_nch
    e_tasks = n_feat_edge * e_nch
    plane = CHUNK * TILE_R

    mesh = plsc.VectorSubcoreMesh(core_axis_name="c", subcore_axis_name="s")

    @functools.partial(
        pl.kernel,
        out_type=(
            jax.ShapeDtypeStruct((n_feat_atom * DIM * n_atom_cols,), jnp.float32),
            jax.ShapeDtypeStruct((n_feat_edge * DIM * n_edge_cols,), jnp.float32),
        ),
        mesh=mesh,
        scratch_types=[
            pltpu.VMEM((atom_rows, DIM), jnp.float32),
            pltpu.VMEM((bond_rows, DIM), jnp.float32),
            pltpu.VMEM((CHUNK,), jnp.int32),
            pltpu.VMEM((2, plane), jnp.float32),
        ],
        compiler_params=pltpu.CompilerParams(
            use_tc_tiling_on_sc=False, needs_layout_passes=False),
    )
    def gather_kernel(atab, xidx, btab, eidx, xout, eout,
                      atab_v, btab_v, idx_v, rows_v):
        wid = lax.axis_index("s") * NC + lax.axis_index("c")
        pltpu.sync_copy(atab, atab_v)
        pltpu.sync_copy(btab, btab_v)

        def run(tab_v, idxs, out, n_tasks, nch, ncols, ntc):
            n_mine = (n_tasks - wid + NW - 1) // NW

            def chunk_body(k, carry):
                task = wid + k * NW
                f = task // nch
                n0 = jnp.minimum((task % nch) * CHUNK, ncols - CHUNK)
                pltpu.sync_copy(idxs.at[pl.ds(f * ncols + n0, CHUNK)], idx_v)

                @plsc.parallel_loop(0, CHUNK // LANES, unroll=2)
                def row_body(j):
                    iv = idx_v[pl.ds(j * LANES, LANES)]
                    off = (j // 8) * (TILE_R * TILE_C) + (j % 8) * LANES
                    vals = [
                        plsc.load_gather(
                            tab_v, [iv, jnp.full((LANES,), d, jnp.int32)])
                        for d in range(DIM)
                    ]
                    for d in range(DIM):
                        rows_v[d // TILE_R,
                               pl.ds(off + (d % TILE_R) * TILE_C, LANES)] = vals[d]
                for tr in range(2):
                    q0 = ((f * 2 + tr) * ntc + n0 // TILE_C) * (TILE_R * TILE_C)
                    pltpu.sync_copy(rows_v.at[tr], out.at[pl.ds(q0, plane)])
                return carry

            lax.fori_loop(0, n_mine, chunk_body, 0)

        run(atab_v, xidx, xout, a_tasks, a_nch, n_atom_cols, a_tc)
        run(btab_v, eidx, eout, e_tasks, e_nch, n_edge_cols, e_tc)

    return gather_kernel


def kernel(x, edge_attr, atom_table, bond_table):
    n_atom, f_atom = x.shape
    n_edge, f_edge = edge_attr.shape
    a_cols = -(-n_atom // TILE_C) * TILE_C
    e_cols = -(-n_edge // TILE_C) * TILE_C

    xt = jnp.pad(x.T.astype(jnp.int32), ((0, 0), (0, a_cols - n_atom)))
    et = jnp.pad(edge_attr.T.astype(jnp.int32), ((0, 0), (0, e_cols - n_edge)))

    gk = _make_gather(a_cols, e_cols, f_atom, f_edge,
                      atom_table.shape[0], bond_table.shape[0])
    xo, eo = gk(atom_table, xt.reshape(-1), bond_table, et.reshape(-1))

    x_emb = (xo.reshape(f_atom, 2, a_cols // TILE_C, TILE_R, TILE_C)
             .transpose(2, 4, 0, 1, 3)
             .reshape(a_cols, f_atom, DIM)[:n_atom])
    e_emb = (eo.reshape(f_edge, 2, e_cols // TILE_C, TILE_R, TILE_C)
             .transpose(2, 4, 0, 1, 3)
             .reshape(e_cols, f_edge, DIM)[:n_edge])
    return (x_emb, e_emb)
